# Initial kernel scaffold; baseline (speedup 1.0000x reference)
import jax
import jax.numpy as jnp
from jax.experimental import pallas as pl


def kernel(anchors, rpn_class_probs, rpn_bbox_deltas):
    raise NotImplementedError("write your pallas kernel here")


# speedup vs baseline: 11.8849x; 11.8849x over previous
import functools

import jax
import jax.numpy as jnp
from jax import lax
from jax.experimental import pallas as pl
from jax.experimental.pallas import tpu as pltpu

_N = 20000
_NPAD = 20480
_B = 8
_K = 6000
_STEPS = 1000
_IMG_W = 512.0
_IMG_H = 512.0
_MIN_SIZE = 16.0
_THR = 0.7
_MAXKEY = 0x3F800000
_BIG = 2 ** 30


def _body(ax1, ay1, ax2, ay2, sc, dx, dy, dw, dh, out_ref,
          bx1, by1, bx2, by2, barea, s_ref):
    wa = ax2[...] - ax1[...]
    ha = ay2[...] - ay1[...]
    cxa = ax1[...] + 0.5 * wa
    cya = ay1[...] + 0.5 * ha
    pcx = dx[...] * wa + cxa
    pcy = dy[...] * ha + cya
    pw = jnp.exp(dw[...]) * wa
    ph = jnp.exp(dh[...]) * ha
    x1 = jnp.clip(pcx - 0.5 * pw, 0.0, _IMG_W - 1.0)
    y1 = jnp.clip(pcy - 0.5 * ph, 0.0, _IMG_H - 1.0)
    x2 = jnp.clip(pcx + 0.5 * pw, 0.0, _IMG_W - 1.0)
    y2 = jnp.clip(pcy + 0.5 * ph, 0.0, _IMG_H - 1.0)
    ws = x2 - x1
    hs = y2 - y1
    valid = (ws >= _MIN_SIZE) & (hs >= _MIN_SIZE)
    bx1[...] = x1
    by1[...] = y1
    bx2[...] = x2
    by2[...] = y2
    barea[...] = ws * hs

    scores = sc[...]
    keys = jnp.where(valid, lax.bitcast_convert_type(scores, jnp.int32),
                     jnp.int32(-1))

    def bs_val(_, lohi):
        lo, hi = lohi
        mid = lo + (hi - lo) // 2
        cnt = jnp.sum((keys > mid).astype(jnp.int32), axis=1, keepdims=True)
        ge = cnt >= _K
        return jnp.where(ge, mid + 1, lo), jnp.where(ge, hi, mid)

    lo0 = jnp.full((_B, 1), -1, jnp.int32)
    hi0 = jnp.full((_B, 1), _MAXKEY, jnp.int32)
    tau, _ = lax.fori_loop(0, 31, bs_val, (lo0, hi0))

    gt = keys > tau
    cnt_gt = jnp.sum(gt.astype(jnp.int32), axis=1, keepdims=True)
    r = _K - cnt_gt
    tie = keys == tau
    iota = lax.broadcasted_iota(jnp.int32, (_B, _NPAD), 1)

    def bs_idx(_, lohi):
        lo, hi = lohi
        mid = lo + (hi - lo) // 2
        cnt = jnp.sum((tie & (iota < mid)).astype(jnp.int32), axis=1,
                      keepdims=True)
        ge = cnt >= r
        return jnp.where(ge, lo, mid + 1), jnp.where(ge, mid, hi)

    jlo0 = jnp.zeros((_B, 1), jnp.int32)
    jhi0 = jnp.full((_B, 1), _NPAD, jnp.int32)
    jcut, _ = lax.fori_loop(0, 15, bs_idx, (jlo0, jhi0))

    cand = gt | (tie & (iota < jcut))
    s_ref[...] = jnp.where(cand & valid, scores, -jnp.inf)

    def step(i, carry):
        s = s_ref[...]
        m = jnp.max(s, axis=1, keepdims=True)
        validb = m > -jnp.inf
        idxs = jnp.where(s == m, iota, _BIG)
        bi = jnp.min(idxs, axis=1, keepdims=True)
        oh = iota == bi
        ohf = oh.astype(jnp.float32)
        vx1 = jnp.sum(ohf * bx1[...], axis=1, keepdims=True)
        vy1 = jnp.sum(ohf * by1[...], axis=1, keepdims=True)
        vx2 = jnp.sum(ohf * bx2[...], axis=1, keepdims=True)
        vy2 = jnp.sum(ohf * by2[...], axis=1, keepdims=True)
        va = (vx2 - vx1) * (vy2 - vy1)
        ix1 = jnp.maximum(vx1, bx1[...])
        iy1 = jnp.maximum(vy1, by1[...])
        ix2 = jnp.minimum(vx2, bx2[...])
        iy2 = jnp.minimum(vy2, by2[...])
        inter = jnp.clip(ix2 - ix1, 0.0) * jnp.clip(iy2 - iy1, 0.0)
        iou = inter / (va + barea[...] - inter + 1e-9)
        s = jnp.where(iou > _THR, -jnp.inf, s)
        s_ref[...] = jnp.where(oh, -jnp.inf, s)
        col = jnp.concatenate([vx1, vy1, vx2, vy2], axis=1)
        col = jnp.where(validb, col, 0.0)
        out_ref[i, :, :] = col
        return carry

    lax.fori_loop(0, _STEPS, step, 0)


def _build_call(interpret=False):
    return pl.pallas_call(
        _body,
        out_shape=jax.ShapeDtypeStruct((_STEPS, _B, 4), jnp.float32),
        scratch_shapes=[pltpu.VMEM((_B, _NPAD), jnp.float32)
                        for _ in range(5)] + [
                        pltpu.VMEM((_B, _NPAD), jnp.float32)],
        interpret=interpret,
    )


@jax.jit
def kernel(anchors, rpn_class_probs, rpn_bbox_deltas):
    pad = _NPAD - _N
    a = jnp.pad(anchors, ((0, pad), (0, 0)))
    sc = jnp.pad(rpn_class_probs[:, :, 1], ((0, 0), (0, pad)))
    d = jnp.pad(rpn_bbox_deltas, ((0, 0), (0, pad), (0, 0)))
    args = (
        a[None, :, 0], a[None, :, 1], a[None, :, 2], a[None, :, 3],
        sc, d[:, :, 0], d[:, :, 1], d[:, :, 2], d[:, :, 3],
    )
    out = _build_call()(*args)
    return out.transpose(1, 0, 2)
